# Initial kernel scaffold; baseline (speedup 1.0000x reference)
import jax
import jax.numpy as jnp
from jax.experimental import pallas as pl


def kernel(x, bond_x, sc_pair_x, angles, mask, bond_idx, sc_idx, angles_idx, W0, b0, W1, b1, g0, beta0, g1, beta1, t):
    raise NotImplementedError("write your pallas kernel here")


# speedup vs baseline: 2.7993x; 2.7993x over previous
import functools

import jax
import jax.numpy as jnp
from jax import lax
from jax.experimental import pallas as pl
from jax.experimental.pallas import tpu as pltpu
from jax.experimental.pallas import tpu_sc as plsc

N_NODES = 10000
D = 128
E = 320000

NC = 2
NS = 16
CHUNK = 80
EDGES_PER_TILE = E // (NC * NS)
N_CHUNKS = EDGES_PER_TILE // CHUNK
ROW_BLK = 632
ROW_LAST = N_NODES - 15 * ROW_BLK

_mesh = plsc.VectorSubcoreMesh(core_axis_name="c", subcore_axis_name="s")


def _zero_agg(zbuf, agg, sid):
    zero16 = jnp.zeros((16,), jnp.float32)

    def zrow(r, carry):
        for c in range(8):
            zbuf[r, pl.ds(c * 16, 16)] = zero16
        return carry

    lax.fori_loop(0, CHUNK, zrow, 0)

    @pl.when(sid < 10)
    def _():
        def zcopy(j, carry):
            off = pl.multiple_of(sid * 1000 + j * CHUNK, 8)
            pltpu.sync_copy(zbuf, agg.at[pl.ds(off, CHUNK)])
            return carry

        lax.fori_loop(0, 12, zcopy, 0)
        off = pl.multiple_of(sid * 1000 + 960, 8)
        pltpu.sync_copy(zbuf.at[pl.ds(0, 40)], agg.at[pl.ds(off, 40)])


def _process_edges(h, src_hbm, dst_hbm, feat_hbm, base, sidx, didx, erow, grow,
                   agg, sem):
    def chunk_body(k, carry):
        off = pl.multiple_of(base + k * CHUNK, 8)
        pltpu.sync_copy(src_hbm.at[pl.ds(off, CHUNK)], sidx)
        pltpu.sync_copy(dst_hbm.at[pl.ds(off, CHUNK)], didx)
        pltpu.sync_copy(feat_hbm.at[pl.ds(off, CHUNK)], erow)
        pltpu.async_copy(h.at[sidx], grow, sem).wait()

        def mrow(r, c2):
            for c in range(8):
                s = pl.ds(c * 16, 16)
                grow[r, s] = grow[r, s] * erow[r, s]
            return c2

        lax.fori_loop(0, CHUNK, mrow, 0)
        pltpu.sync_copy(grow, agg.at[didx], add=True)
        return carry

    lax.fori_loop(0, N_CHUNKS, chunk_body, 0)


def _write_out(agg, out, cid, sid):
    @pl.when(sid < 15)
    def _():
        rows = pl.ds(sid * ROW_BLK, ROW_BLK)
        pltpu.sync_copy(agg.at[rows], out.at[cid, rows])

    @pl.when(sid == 15)
    def _():
        rows = pl.ds(15 * ROW_BLK, ROW_LAST)
        pltpu.sync_copy(agg.at[rows], out.at[cid, rows])


_SC_SCRATCH = [
    pltpu.VMEM((CHUNK,), jnp.int32),
    pltpu.VMEM((CHUNK,), jnp.int32),
    pltpu.VMEM((CHUNK, D), jnp.float32),
    pltpu.VMEM((CHUNK, D), jnp.float32),
    pltpu.VMEM_SHARED((N_NODES, D), jnp.float32),
    pltpu.SemaphoreType.DMA,
]


@functools.partial(
    pl.kernel,
    out_type=jax.ShapeDtypeStruct((NC, N_NODES, D), jnp.float32),
    mesh=_mesh,
    scratch_types=_SC_SCRATCH,
)
def _sc_message2(h, src0, dst0, feat0, src1, dst1, feat1, out,
                 sidx, didx, erow, grow, agg, sem):
    cid = lax.axis_index("c")
    sid = lax.axis_index("s")
    _zero_agg(grow, agg, sid)
    plsc.subcore_barrier()
    base = cid * (E // NC) + sid * EDGES_PER_TILE
    _process_edges(h, src0, dst0, feat0, base, sidx, didx, erow, grow, agg, sem)
    _process_edges(h, src1, dst1, feat1, base, sidx, didx, erow, grow, agg, sem)
    plsc.subcore_barrier()
    _write_out(agg, out, cid, sid)


@functools.partial(
    pl.kernel,
    out_type=jax.ShapeDtypeStruct((NC, N_NODES, D), jnp.float32),
    mesh=_mesh,
    scratch_types=_SC_SCRATCH,
)
def _sc_message1(h, src0, dst0, feat0, out,
                 sidx, didx, erow, grow, agg, sem):
    cid = lax.axis_index("c")
    sid = lax.axis_index("s")
    _zero_agg(grow, agg, sid)
    plsc.subcore_barrier()
    base = cid * (E // NC) + sid * EDGES_PER_TILE
    _process_edges(h, src0, dst0, feat0, base, sidx, didx, erow, grow, agg, sem)
    plsc.subcore_barrier()
    _write_out(agg, out, cid, sid)


_ROW_BLK = 1000
_GRID = N_NODES // _ROW_BLK


def _ln(x, g, b):
    mu = jnp.mean(x, axis=-1, keepdims=True)
    var = jnp.mean((x - mu) ** 2, axis=-1, keepdims=True)
    return (x - mu) / jnp.sqrt(var + 1e-5) * g + b


def _ln_body(x_ref, g_ref, b_ref, o_ref):
    o_ref[...] = _ln(x_ref[...], g_ref[...], b_ref[...])


def _combine1_body(x_ref, a_ref, w_ref, b_ref, g_ref, beta_ref,
                   x1_ref, h2_ref):
    s = a_ref[0] + a_ref[1]
    y = lax.dot_general(s, w_ref[...], (((1,), (1,)), ((), ())),
                        preferred_element_type=jnp.float32)
    x1 = x_ref[...] + y + b_ref[...]
    x1_ref[...] = x1
    h2_ref[...] = _ln(x1, g_ref[...], beta_ref[...])


def _combine2_body(x_ref, a_ref, w_ref, b_ref, x2_ref):
    s = a_ref[0] + a_ref[1]
    y = lax.dot_general(s, w_ref[...], (((1,), (1,)), ((), ())),
                        preferred_element_type=jnp.float32)
    x2_ref[...] = x_ref[...] + y + b_ref[...]


_row_spec = pl.BlockSpec((_ROW_BLK, D), lambda i: (i, 0))
_agg_spec = pl.BlockSpec((NC, _ROW_BLK, D), lambda i: (0, i, 0))
_vec_spec = pl.BlockSpec((1, D), lambda i: (0, 0))
_w_spec = pl.BlockSpec((D, D), lambda i: (0, 0))

_ln_call = pl.pallas_call(
    _ln_body,
    grid=(_GRID,),
    in_specs=[_row_spec, _vec_spec, _vec_spec],
    out_specs=_row_spec,
    out_shape=jax.ShapeDtypeStruct((N_NODES, D), jnp.float32),
)

_combine1_call = pl.pallas_call(
    _combine1_body,
    grid=(_GRID,),
    in_specs=[_row_spec, _agg_spec, _w_spec, _vec_spec, _vec_spec, _vec_spec],
    out_specs=[_row_spec, _row_spec],
    out_shape=[jax.ShapeDtypeStruct((N_NODES, D), jnp.float32),
               jax.ShapeDtypeStruct((N_NODES, D), jnp.float32)],
)

_combine2_call = pl.pallas_call(
    _combine2_body,
    grid=(_GRID,),
    in_specs=[_row_spec, _agg_spec, _w_spec, _vec_spec],
    out_specs=_row_spec,
    out_shape=jax.ShapeDtypeStruct((N_NODES, D), jnp.float32),
)


def kernel(x, bond_x, sc_pair_x, angles, mask, bond_idx, sc_idx, angles_idx,
           W0, b0, W1, b1, g0, beta0, g1, beta1, t):
    del mask
    g0r, beta0r = g0.reshape(1, D), beta0.reshape(1, D)
    g1r, beta1r = g1.reshape(1, D), beta1.reshape(1, D)
    b0r, b1r = b0.reshape(1, D), b1.reshape(1, D)

    h = _ln_call(x, g0r, beta0r)
    agg = _sc_message2(h, bond_idx[0], bond_idx[1], bond_x,
                       angles_idx[0], angles_idx[1], angles)
    x1, h2 = _combine1_call(x, agg, W0, b0r, g1r, beta1r)
    agg2 = _sc_message1(h2, sc_idx[0], sc_idx[1], sc_pair_x)
    x2 = _combine2_call(x1, agg2, W1, b1r)
    return x2 + (jnp.asarray(t) * 0).astype(x2.dtype)


# speedup vs baseline: 6.6113x; 2.3617x over previous
import functools

import jax
import jax.numpy as jnp
from jax import lax
from jax.experimental import pallas as pl
from jax.experimental.pallas import tpu as pltpu
from jax.experimental.pallas import tpu_sc as plsc

N_NODES = 10000
D = 128
E = 320000

NC = 2
NS = 16
CHUNK = 80
EDGES_PER_TILE = E // (NC * NS)
N_CHUNKS = EDGES_PER_TILE // CHUNK
GROUP_CHUNKS = 25
GROUP = GROUP_CHUNKS * CHUNK

_mesh = plsc.VectorSubcoreMesh(core_axis_name="c", subcore_axis_name="s")


def _zero_agg(zbuf, agg, sid):
    zero16 = jnp.zeros((16,), jnp.float32)

    def zrow(r, carry):
        for c in range(8):
            zbuf[r, pl.ds(c * 16, 16)] = zero16
        return carry

    lax.fori_loop(0, CHUNK, zrow, 0)

    @pl.when(sid < 10)
    def _():
        def zcopy(j, carry):
            off = pl.multiple_of(sid * 1000 + j * CHUNK, 8)
            pltpu.sync_copy(zbuf, agg.at[pl.ds(off, CHUNK)])
            return carry

        lax.fori_loop(0, 12, zcopy, 0)
        off = pl.multiple_of(sid * 1000 + 960, 8)
        pltpu.sync_copy(zbuf.at[pl.ds(0, 40)], agg.at[pl.ds(off, 40)])


def _write_out(agg, out, cid, sid):
    @pl.when(sid < 15)
    def _():
        rows = pl.ds(sid * 632, 632)
        pltpu.sync_copy(agg.at[rows], out.at[cid, rows])

    @pl.when(sid == 15)
    def _():
        rows = pl.ds(15 * 632, 520)
        pltpu.sync_copy(agg.at[rows], out.at[cid, rows])


def _run_set(h, src_hbm, dst_hbm, feat_hbm, base, bufs):
    (e0, e1, g0, g1, sbuf, dbuf, d0, d1, agg,
     sf0, sf1, sg0, sg1, ss0, ss1) = bufs

    def stage_group(goff):
        pltpu.sync_copy(src_hbm.at[pl.ds(goff, GROUP)], sbuf)
        pltpu.sync_copy(dst_hbm.at[pl.ds(goff, GROUP)], dbuf)

    def feat_copy(k, ebuf, sem):
        off = pl.multiple_of(base + k * CHUNK, 8)
        return pltpu.make_async_copy(feat_hbm.at[pl.ds(off, CHUNK)], ebuf, sem)

    def gather_copy(k, gbuf, sem):
        j = lax.rem(k, GROUP_CHUNKS)
        idx = sbuf.at[pl.ds(j * CHUNK, CHUNK)]
        return pltpu.make_async_copy(h.at[idx], gbuf, sem)

    def step(k, cur_e, cur_g, nxt_e, nxt_g, cur_d, prev_d,
             sf_c, sg_c, sf_n, sg_n, ss_c, ss_p, last):
        j = lax.rem(k, GROUP_CHUNKS)
        feat_copy(k, cur_e, sf_c).wait()
        gather_copy(k, cur_g, sg_c).wait()

        for c in range(CHUNK // 16):
            cur_d[pl.ds(c * 16, 16)] = dbuf[pl.ds(j * CHUNK + c * 16, 16)]

        if not last:
            @pl.when(j == GROUP_CHUNKS - 1)
            def _():
                stage_group(pl.multiple_of(base + (k + 1) * CHUNK, 8))

            @pl.when(k >= 1)
            def _():
                pltpu.make_async_copy(nxt_g, agg.at[prev_d], ss_p).wait()

            feat_copy(k + 1, nxt_e, sf_n).start()
            gather_copy(k + 1, nxt_g, sg_n).start()

        def mrow(r, carry):
            for c in range(8):
                s = pl.ds(c * 16, 16)
                cur_g[r, s] = cur_g[r, s] * cur_e[r, s]
            return carry

        lax.fori_loop(0, CHUNK, mrow, 0)
        pltpu.async_copy(cur_g, agg.at[cur_d], ss_c, add=True)

    stage_group(pl.multiple_of(base, 8))
    feat_copy(0, e0, sf0).start()
    gather_copy(0, g0, sg0).start()

    def pair(k2, carry):
        k = k2 * 2
        step(k, e0, g0, e1, g1, d0, d1, sf0, sg0, sf1, sg1, ss0, ss1, False)
        step(k + 1, e1, g1, e0, g0, d1, d0, sf1, sg1, sf0, sg0, ss1, ss0,
             False)
        return carry

    lax.fori_loop(0, (N_CHUNKS - 1) // 2, pair, 0)
    step(N_CHUNKS - 1, e0, g0, e1, g1, d0, d1, sf0, sg0, sf1, sg1, ss0, ss1,
         True)
    pltpu.make_async_copy(g1, agg.at[d1], ss1).wait()
    pltpu.make_async_copy(g0, agg.at[d0], ss0).wait()


_SC_SCRATCH = [
    pltpu.VMEM((CHUNK, D), jnp.float32),
    pltpu.VMEM((CHUNK, D), jnp.float32),
    pltpu.VMEM((CHUNK, D), jnp.float32),
    pltpu.VMEM((CHUNK, D), jnp.float32),
    pltpu.VMEM((GROUP,), jnp.int32),
    pltpu.VMEM((GROUP,), jnp.int32),
    pltpu.VMEM((CHUNK,), jnp.int32),
    pltpu.VMEM((CHUNK,), jnp.int32),
    pltpu.VMEM_SHARED((N_NODES, D), jnp.float32),
    pltpu.SemaphoreType.DMA,
    pltpu.SemaphoreType.DMA,
    pltpu.SemaphoreType.DMA,
    pltpu.SemaphoreType.DMA,
    pltpu.SemaphoreType.DMA,
    pltpu.SemaphoreType.DMA,
]


@functools.partial(
    pl.kernel,
    out_type=jax.ShapeDtypeStruct((NC, N_NODES, D), jnp.float32),
    mesh=_mesh,
    scratch_types=_SC_SCRATCH,
)
def _sc_message2(h, src0, dst0, feat0, src1, dst1, feat1, out, *bufs):
    cid = lax.axis_index("c")
    sid = lax.axis_index("s")
    agg = bufs[8]
    _zero_agg(bufs[2], agg, sid)
    plsc.subcore_barrier()
    base = cid * (E // NC) + sid * EDGES_PER_TILE
    _run_set(h, src0, dst0, feat0, base, bufs)
    _run_set(h, src1, dst1, feat1, base, bufs)
    plsc.subcore_barrier()
    _write_out(agg, out, cid, sid)


@functools.partial(
    pl.kernel,
    out_type=jax.ShapeDtypeStruct((NC, N_NODES, D), jnp.float32),
    mesh=_mesh,
    scratch_types=_SC_SCRATCH,
)
def _sc_message1(h, src0, dst0, feat0, out, *bufs):
    cid = lax.axis_index("c")
    sid = lax.axis_index("s")
    agg = bufs[8]
    _zero_agg(bufs[2], agg, sid)
    plsc.subcore_barrier()
    base = cid * (E // NC) + sid * EDGES_PER_TILE
    _run_set(h, src0, dst0, feat0, base, bufs)
    plsc.subcore_barrier()
    _write_out(agg, out, cid, sid)


_ROW_BLK = 1000
_GRID = N_NODES // _ROW_BLK


def _ln(x, g, b):
    mu = jnp.mean(x, axis=-1, keepdims=True)
    var = jnp.mean((x - mu) ** 2, axis=-1, keepdims=True)
    return (x - mu) / jnp.sqrt(var + 1e-5) * g + b


def _ln_body(x_ref, g_ref, b_ref, o_ref):
    o_ref[...] = _ln(x_ref[...], g_ref[...], b_ref[...])


def _combine1_body(x_ref, a_ref, w_ref, b_ref, g_ref, beta_ref,
                   x1_ref, h2_ref):
    s = a_ref[0] + a_ref[1]
    y = lax.dot_general(s, w_ref[...], (((1,), (1,)), ((), ())),
                        preferred_element_type=jnp.float32)
    x1 = x_ref[...] + y + b_ref[...]
    x1_ref[...] = x1
    h2_ref[...] = _ln(x1, g_ref[...], beta_ref[...])


def _combine2_body(x_ref, a_ref, w_ref, b_ref, x2_ref):
    s = a_ref[0] + a_ref[1]
    y = lax.dot_general(s, w_ref[...], (((1,), (1,)), ((), ())),
                        preferred_element_type=jnp.float32)
    x2_ref[...] = x_ref[...] + y + b_ref[...]


_row_spec = pl.BlockSpec((_ROW_BLK, D), lambda i: (i, 0))
_agg_spec = pl.BlockSpec((NC, _ROW_BLK, D), lambda i: (0, i, 0))
_vec_spec = pl.BlockSpec((1, D), lambda i: (0, 0))
_w_spec = pl.BlockSpec((D, D), lambda i: (0, 0))

_ln_call = pl.pallas_call(
    _ln_body,
    grid=(_GRID,),
    in_specs=[_row_spec, _vec_spec, _vec_spec],
    out_specs=_row_spec,
    out_shape=jax.ShapeDtypeStruct((N_NODES, D), jnp.float32),
)

_combine1_call = pl.pallas_call(
    _combine1_body,
    grid=(_GRID,),
    in_specs=[_row_spec, _agg_spec, _w_spec, _vec_spec, _vec_spec, _vec_spec],
    out_specs=[_row_spec, _row_spec],
    out_shape=[jax.ShapeDtypeStruct((N_NODES, D), jnp.float32),
               jax.ShapeDtypeStruct((N_NODES, D), jnp.float32)],
)

_combine2_call = pl.pallas_call(
    _combine2_body,
    grid=(_GRID,),
    in_specs=[_row_spec, _agg_spec, _w_spec, _vec_spec],
    out_specs=_row_spec,
    out_shape=jax.ShapeDtypeStruct((N_NODES, D), jnp.float32),
)


def kernel(x, bond_x, sc_pair_x, angles, mask, bond_idx, sc_idx, angles_idx,
           W0, b0, W1, b1, g0, beta0, g1, beta1, t):
    del mask
    g0r, beta0r = g0.reshape(1, D), beta0.reshape(1, D)
    g1r, beta1r = g1.reshape(1, D), beta1.reshape(1, D)
    b0r, b1r = b0.reshape(1, D), b1.reshape(1, D)

    h = _ln_call(x, g0r, beta0r)
    agg = _sc_message2(h, bond_idx[0], bond_idx[1], bond_x,
                       angles_idx[0], angles_idx[1], angles)
    x1, h2 = _combine1_call(x, agg, W0, b0r, g1r, beta1r)
    agg2 = _sc_message1(h2, sc_idx[0], sc_idx[1], sc_pair_x)
    x2 = _combine2_call(x1, agg2, W1, b1r)
    return x2 + (jnp.asarray(t) * 0).astype(x2.dtype)


# speedup vs baseline: 6.6677x; 1.0085x over previous
import functools

import jax
import jax.numpy as jnp
from jax import lax
from jax.experimental import pallas as pl
from jax.experimental.pallas import tpu as pltpu
from jax.experimental.pallas import tpu_sc as plsc

N_NODES = 10000
D = 128
E = 320000

NC = 2
NS = 16
CHUNK = 80
EDGES_PER_TILE = E // (NC * NS)
N_CHUNKS = EDGES_PER_TILE // CHUNK
GROUP_CHUNKS = 25
GROUP = GROUP_CHUNKS * CHUNK

_mesh = plsc.VectorSubcoreMesh(core_axis_name="c", subcore_axis_name="s")


def _zero_agg(zbuf, agg, sid):
    zero16 = jnp.zeros((16,), jnp.float32)

    def zrow(r, carry):
        for c in range(8):
            zbuf[r, pl.ds(c * 16, 16)] = zero16
        return carry

    lax.fori_loop(0, CHUNK, zrow, 0)

    @pl.when(sid < 10)
    def _():
        def zcopy(j, carry):
            off = pl.multiple_of(sid * 1000 + j * CHUNK, 8)
            pltpu.sync_copy(zbuf, agg.at[pl.ds(off, CHUNK)])
            return carry

        lax.fori_loop(0, 12, zcopy, 0)
        off = pl.multiple_of(sid * 1000 + 960, 8)
        pltpu.sync_copy(zbuf.at[pl.ds(0, 40)], agg.at[pl.ds(off, 40)])


def _write_out(agg, out, cid, sid):
    @pl.when(sid < 15)
    def _():
        rows = pl.ds(sid * 632, 632)
        pltpu.sync_copy(agg.at[rows], out.at[cid, rows])

    @pl.when(sid == 15)
    def _():
        rows = pl.ds(15 * 632, 520)
        pltpu.sync_copy(agg.at[rows], out.at[cid, rows])


def _run_set(h, src_hbm, dst_hbm, feat_hbm, base, bufs):
    (e0, e1, g0, g1, sbuf, dbuf, d0, d1, agg,
     sf0, sf1, sg0, sg1, ss0, ss1, si) = bufs

    N_GROUPS = N_CHUNKS // GROUP_CHUNKS

    def stage_copies(g):
        goff = pl.multiple_of(base + g * GROUP, 8)
        p = lax.rem(g, 2)
        return (pltpu.make_async_copy(src_hbm.at[pl.ds(goff, GROUP)],
                                      sbuf.at[pl.ds(p * GROUP, GROUP)], si),
                pltpu.make_async_copy(dst_hbm.at[pl.ds(goff, GROUP)],
                                      dbuf.at[pl.ds(p * GROUP, GROUP)], si))

    def stage_start(g):
        a, b = stage_copies(g)
        a.start()
        b.start()

    def stage_wait(g):
        a, b = stage_copies(g)
        a.wait()
        b.wait()

    def feat_copy(k, ebuf, sem):
        off = pl.multiple_of(base + k * CHUNK, 8)
        return pltpu.make_async_copy(feat_hbm.at[pl.ds(off, CHUNK)], ebuf, sem)

    def gather_copy(k, gbuf, sem):
        j = lax.rem(k, GROUP_CHUNKS)
        p = lax.rem(k // GROUP_CHUNKS, 2)
        idx = sbuf.at[pl.ds(p * GROUP + j * CHUNK, CHUNK)]
        return pltpu.make_async_copy(h.at[idx], gbuf, sem)

    def step(k, cur_e, cur_g, nxt_e, nxt_g, cur_d, prev_d,
             sf_c, sg_c, sf_n, sg_n, ss_c, ss_p, last):
        j = lax.rem(k, GROUP_CHUNKS)
        p = lax.rem(k // GROUP_CHUNKS, 2)
        feat_copy(k, cur_e, sf_c).wait()
        gather_copy(k, cur_g, sg_c).wait()

        for c in range(CHUNK // 16):
            cur_d[pl.ds(c * 16, 16)] = dbuf[pl.ds(p * GROUP + j * CHUNK + c * 16, 16)]

        if not last:
            @pl.when(j == GROUP_CHUNKS - 1)
            def _():
                g_next = (k + 1) // GROUP_CHUNKS
                stage_wait(g_next)

                @pl.when(k + 1 < (N_GROUPS - 1) * GROUP_CHUNKS)
                def _():
                    stage_start(g_next + 1)

            @pl.when(k >= 1)
            def _():
                pltpu.make_async_copy(nxt_g, agg.at[prev_d], ss_p).wait()

            feat_copy(k + 1, nxt_e, sf_n).start()
            gather_copy(k + 1, nxt_g, sg_n).start()

        @plsc.parallel_loop(0, CHUNK, step=1, unroll=4)
        def mrow(r):
            for c in range(8):
                s = pl.ds(c * 16, 16)
                cur_g[r, s] = cur_g[r, s] * cur_e[r, s]

        pltpu.async_copy(cur_g, agg.at[cur_d], ss_c, add=True)

    stage_start(0)
    stage_wait(0)
    stage_start(1)
    feat_copy(0, e0, sf0).start()
    gather_copy(0, g0, sg0).start()

    def pair(k2, carry):
        k = k2 * 2
        step(k, e0, g0, e1, g1, d0, d1, sf0, sg0, sf1, sg1, ss0, ss1, False)
        step(k + 1, e1, g1, e0, g0, d1, d0, sf1, sg1, sf0, sg0, ss1, ss0,
             False)
        return carry

    lax.fori_loop(0, (N_CHUNKS - 1) // 2, pair, 0)
    step(N_CHUNKS - 1, e0, g0, e1, g1, d0, d1, sf0, sg0, sf1, sg1, ss0, ss1,
         True)
    pltpu.make_async_copy(g1, agg.at[d1], ss1).wait()
    pltpu.make_async_copy(g0, agg.at[d0], ss0).wait()


_SC_SCRATCH = [
    pltpu.VMEM((CHUNK, D), jnp.float32),
    pltpu.VMEM((CHUNK, D), jnp.float32),
    pltpu.VMEM((CHUNK, D), jnp.float32),
    pltpu.VMEM((CHUNK, D), jnp.float32),
    pltpu.VMEM((2 * GROUP,), jnp.int32),
    pltpu.VMEM((2 * GROUP,), jnp.int32),
    pltpu.VMEM((CHUNK,), jnp.int32),
    pltpu.VMEM((CHUNK,), jnp.int32),
    pltpu.VMEM_SHARED((N_NODES, D), jnp.float32),
    pltpu.SemaphoreType.DMA,
    pltpu.SemaphoreType.DMA,
    pltpu.SemaphoreType.DMA,
    pltpu.SemaphoreType.DMA,
    pltpu.SemaphoreType.DMA,
    pltpu.SemaphoreType.DMA,
    pltpu.SemaphoreType.DMA,
]


@functools.partial(
    pl.kernel,
    out_type=jax.ShapeDtypeStruct((NC, N_NODES, D), jnp.float32),
    mesh=_mesh,
    scratch_types=_SC_SCRATCH,
)
def _sc_message2(h, src0, dst0, feat0, src1, dst1, feat1, out, *bufs):
    cid = lax.axis_index("c")
    sid = lax.axis_index("s")
    agg = bufs[8]
    _zero_agg(bufs[2], agg, sid)
    plsc.subcore_barrier()
    base = cid * (E // NC) + sid * EDGES_PER_TILE
    _run_set(h, src0, dst0, feat0, base, bufs)
    _run_set(h, src1, dst1, feat1, base, bufs)
    plsc.subcore_barrier()
    _write_out(agg, out, cid, sid)


@functools.partial(
    pl.kernel,
    out_type=jax.ShapeDtypeStruct((NC, N_NODES, D), jnp.float32),
    mesh=_mesh,
    scratch_types=_SC_SCRATCH,
)
def _sc_message1(h, src0, dst0, feat0, out, *bufs):
    cid = lax.axis_index("c")
    sid = lax.axis_index("s")
    agg = bufs[8]
    _zero_agg(bufs[2], agg, sid)
    plsc.subcore_barrier()
    base = cid * (E // NC) + sid * EDGES_PER_TILE
    _run_set(h, src0, dst0, feat0, base, bufs)
    plsc.subcore_barrier()
    _write_out(agg, out, cid, sid)


_ROW_BLK = 1000
_GRID = N_NODES // _ROW_BLK


def _ln(x, g, b):
    mu = jnp.mean(x, axis=-1, keepdims=True)
    var = jnp.mean((x - mu) ** 2, axis=-1, keepdims=True)
    return (x - mu) / jnp.sqrt(var + 1e-5) * g + b


def _ln_body(x_ref, g_ref, b_ref, o_ref):
    o_ref[...] = _ln(x_ref[...], g_ref[...], b_ref[...])


def _combine1_body(x_ref, a_ref, w_ref, b_ref, g_ref, beta_ref,
                   x1_ref, h2_ref):
    s = a_ref[0] + a_ref[1]
    y = lax.dot_general(s, w_ref[...], (((1,), (1,)), ((), ())),
                        preferred_element_type=jnp.float32)
    x1 = x_ref[...] + y + b_ref[...]
    x1_ref[...] = x1
    h2_ref[...] = _ln(x1, g_ref[...], beta_ref[...])


def _combine2_body(x_ref, a_ref, w_ref, b_ref, x2_ref):
    s = a_ref[0] + a_ref[1]
    y = lax.dot_general(s, w_ref[...], (((1,), (1,)), ((), ())),
                        preferred_element_type=jnp.float32)
    x2_ref[...] = x_ref[...] + y + b_ref[...]


_row_spec = pl.BlockSpec((_ROW_BLK, D), lambda i: (i, 0))
_agg_spec = pl.BlockSpec((NC, _ROW_BLK, D), lambda i: (0, i, 0))
_vec_spec = pl.BlockSpec((1, D), lambda i: (0, 0))
_w_spec = pl.BlockSpec((D, D), lambda i: (0, 0))

_ln_call = pl.pallas_call(
    _ln_body,
    grid=(_GRID,),
    in_specs=[_row_spec, _vec_spec, _vec_spec],
    out_specs=_row_spec,
    out_shape=jax.ShapeDtypeStruct((N_NODES, D), jnp.float32),
)

_combine1_call = pl.pallas_call(
    _combine1_body,
    grid=(_GRID,),
    in_specs=[_row_spec, _agg_spec, _w_spec, _vec_spec, _vec_spec, _vec_spec],
    out_specs=[_row_spec, _row_spec],
    out_shape=[jax.ShapeDtypeStruct((N_NODES, D), jnp.float32),
               jax.ShapeDtypeStruct((N_NODES, D), jnp.float32)],
)

_combine2_call = pl.pallas_call(
    _combine2_body,
    grid=(_GRID,),
    in_specs=[_row_spec, _agg_spec, _w_spec, _vec_spec],
    out_specs=_row_spec,
    out_shape=jax.ShapeDtypeStruct((N_NODES, D), jnp.float32),
)


def kernel(x, bond_x, sc_pair_x, angles, mask, bond_idx, sc_idx, angles_idx,
           W0, b0, W1, b1, g0, beta0, g1, beta1, t):
    del mask
    g0r, beta0r = g0.reshape(1, D), beta0.reshape(1, D)
    g1r, beta1r = g1.reshape(1, D), beta1.reshape(1, D)
    b0r, b1r = b0.reshape(1, D), b1.reshape(1, D)

    h = _ln_call(x, g0r, beta0r)
    agg = _sc_message2(h, bond_idx[0], bond_idx[1], bond_x,
                       angles_idx[0], angles_idx[1], angles)
    x1, h2 = _combine1_call(x, agg, W0, b0r, g1r, beta1r)
    agg2 = _sc_message1(h2, sc_idx[0], sc_idx[1], sc_pair_x)
    x2 = _combine2_call(x1, agg2, W1, b1r)
    return x2 + (jnp.asarray(t) * 0).astype(x2.dtype)
